# baseline (speedup 1.0000x reference)
import functools
import jax
import jax.numpy as jnp
from jax import lax
from jax.experimental import pallas as pl
from jax.experimental.pallas import tpu as pltpu, tpu_sc as plsc

B, T, D = 4, 8192, 1024
NC, NS = 2, 16
NW = NC * NS
TW = T // NW
BTR = 4
NB = TW // BTR
L = 16
NX = 4
NP = 3
PF = 2


def _sc_add(x_hbm, pe_hbm, out_hbm, pe_v, x_v, *sems):
    sem_pe = sems[0:NP]
    sem_x = sems[NP:NP + NX]
    sem_o = sems[NP + NX:NP + 2 * NX]
    wid = lax.axis_index("s") * NC + lax.axis_index("c")
    base = wid * TW

    def issue_in(blk):
        t0 = base + blk * BTR
        dpe = pltpu.async_copy(pe_hbm.at[pl.ds(t0, BTR), :],
                               pe_v.at[blk % NP], sem_pe[blk % NP])
        dx = pltpu.async_copy(x_hbm.at[:, pl.ds(t0, BTR), :],
                              x_v.at[blk % NX], sem_x[blk % NX])
        return dpe, dx

    def compute(blk):
        pe_s = pe_v.at[blk % NP]
        x_s = x_v.at[blk % NX]

        def row(r, _):
            def chunk(j, _):
                o = j * L
                pe_vec = pe_s[r, pl.ds(o, L)]
                for b in range(B):
                    plsc.addupdate(x_s.at[b, r, pl.ds(o, L)], pe_vec)
                return 0

            lax.fori_loop(0, D // L, chunk, 0, unroll=8)
            return 0

        lax.fori_loop(0, BTR, row, 0)

    descs_in = {b: issue_in(b) for b in range(min(PF, NB))}
    descs_out = {}
    for blk in range(NB):
        nxt = blk + PF
        if nxt < NB:
            prev_user = nxt - NX
            if prev_user >= 0:
                descs_out.pop(prev_user).wait()
            descs_in[nxt] = issue_in(nxt)
        dpe, dx = descs_in.pop(blk)
        dpe.wait()
        dx.wait()
        compute(blk)
        t0 = base + blk * BTR
        descs_out[blk] = pltpu.async_copy(
            x_v.at[blk % NX], out_hbm.at[:, pl.ds(t0, BTR), :],
            sem_o[blk % NX])
    for blk in sorted(descs_out):
        descs_out.pop(blk).wait()


_mesh = plsc.VectorSubcoreMesh(core_axis_name="c", subcore_axis_name="s")

_sc_call = functools.partial(
    pl.kernel,
    out_type=jax.ShapeDtypeStruct((B, T, D), jnp.float32),
    mesh=_mesh,
    scratch_types=(
        [pltpu.VMEM((NP, BTR, D), jnp.float32),
         pltpu.VMEM((NX, B, BTR, D), jnp.float32)]
        + [pltpu.SemaphoreType.DMA] * (NP + 2 * NX)
    ),
)(_sc_add)


def kernel(x, pe_table):
    return _sc_call(x, pe_table[:T])


# speedup vs baseline: 1.0027x; 1.0027x over previous
import functools
import jax
import jax.numpy as jnp
from jax import lax
from jax.experimental import pallas as pl
from jax.experimental.pallas import tpu as pltpu, tpu_sc as plsc

B, T, D = 4, 8192, 1024
NC, NS = 2, 16
NW = NC * NS
TW = T // NW
BTR = 8
NB = TW // BTR
L = 16
NX = 3
NP = 2
PF = 1


def _sc_add(x_hbm, pe_hbm, out_hbm, pe_v, x_v, *sems):
    sem_pe = sems[0:NP]
    sem_x = sems[NP:NP + NX]
    sem_o = sems[NP + NX:NP + 2 * NX]
    wid = lax.axis_index("s") * NC + lax.axis_index("c")
    base = wid * TW

    def issue_in(blk):
        t0 = base + blk * BTR
        dpe = pltpu.async_copy(pe_hbm.at[pl.ds(t0, BTR), :],
                               pe_v.at[blk % NP], sem_pe[blk % NP])
        dx = pltpu.async_copy(x_hbm.at[:, pl.ds(t0, BTR), :],
                              x_v.at[blk % NX], sem_x[blk % NX])
        return dpe, dx

    def compute(blk):
        pe_s = pe_v.at[blk % NP]
        x_s = x_v.at[blk % NX]

        def row(r, _):
            def chunk(j, _):
                o = j * L
                pe_vec = pe_s[r, pl.ds(o, L)]
                for b in range(B):
                    plsc.addupdate(x_s.at[b, r, pl.ds(o, L)], pe_vec)
                return 0

            lax.fori_loop(0, D // L, chunk, 0, unroll=16)
            return 0

        lax.fori_loop(0, BTR, row, 0)

    descs_in = {b: issue_in(b) for b in range(min(PF, NB))}
    descs_out = {}
    for blk in range(NB):
        nxt = blk + PF
        if nxt < NB:
            prev_user = nxt - NX
            if prev_user >= 0:
                descs_out.pop(prev_user).wait()
            descs_in[nxt] = issue_in(nxt)
        dpe, dx = descs_in.pop(blk)
        dpe.wait()
        dx.wait()
        compute(blk)
        t0 = base + blk * BTR
        descs_out[blk] = pltpu.async_copy(
            x_v.at[blk % NX], out_hbm.at[:, pl.ds(t0, BTR), :],
            sem_o[blk % NX])
    for blk in sorted(descs_out):
        descs_out.pop(blk).wait()


_mesh = plsc.VectorSubcoreMesh(core_axis_name="c", subcore_axis_name="s")

_sc_call = functools.partial(
    pl.kernel,
    out_type=jax.ShapeDtypeStruct((B, T, D), jnp.float32),
    mesh=_mesh,
    scratch_types=(
        [pltpu.VMEM((NP, BTR, D), jnp.float32),
         pltpu.VMEM((NX, B, BTR, D), jnp.float32)]
        + [pltpu.SemaphoreType.DMA] * (NP + 2 * NX)
    ),
)(_sc_add)


def kernel(x, pe_table):
    return _sc_call(x, pe_table[:T])


# speedup vs baseline: 1.0305x; 1.0278x over previous
import functools
import jax
import jax.numpy as jnp
from jax import lax
from jax.experimental import pallas as pl
from jax.experimental.pallas import tpu as pltpu, tpu_sc as plsc

B, T, D = 4, 8192, 1024
NC, NS = 2, 16
NW = NC * NS
TW = T // NW
BTR = 8
NB = TW // BTR
L = 16
NX = 3
NP = 2
PF = 1


def _sc_add(x_hbm, pe_hbm, out_hbm, pe_v, x_v, *sems):
    sem_pe = sems[0:NP]
    sem_x = sems[NP:NP + NX]
    sem_o = sems[NP + NX:NP + 2 * NX]
    wid = lax.axis_index("s") * NC + lax.axis_index("c")
    base = wid * TW

    def issue_in(blk):
        t0 = base + blk * BTR
        dpe = pltpu.async_copy(pe_hbm.at[pl.ds(t0, BTR), :],
                               pe_v.at[blk % NP], sem_pe[blk % NP])
        dx = pltpu.async_copy(x_hbm.at[:, pl.ds(t0, BTR), :],
                              x_v.at[blk % NX], sem_x[blk % NX])
        return dpe, dx

    def compute(blk):
        pe_s = pe_v.at[blk % NP]
        x_s = x_v.at[blk % NX]

        def row(r, _):
            def chunk(j, _):
                o = j * L
                pe_vec = pe_s[r, pl.ds(o, L)]
                for b in range(B):
                    plsc.addupdate(x_s.at[b, r, pl.ds(o, L)], pe_vec)
                return 0

            lax.fori_loop(0, D // L, chunk, 0, unroll=8)
            return 0

        lax.fori_loop(0, BTR, row, 0)

    descs_in = {b: issue_in(b) for b in range(min(PF, NB))}
    descs_out = {}
    for blk in range(NB):
        nxt = blk + PF
        if nxt < NB:
            prev_user = nxt - NX
            if prev_user >= 0:
                descs_out.pop(prev_user).wait()
            descs_in[nxt] = issue_in(nxt)
        dpe, dx = descs_in.pop(blk)
        dpe.wait()
        dx.wait()
        compute(blk)
        t0 = base + blk * BTR
        descs_out[blk] = pltpu.async_copy(
            x_v.at[blk % NX], out_hbm.at[:, pl.ds(t0, BTR), :],
            sem_o[blk % NX])
    for blk in sorted(descs_out):
        descs_out.pop(blk).wait()


_mesh = plsc.VectorSubcoreMesh(core_axis_name="c", subcore_axis_name="s")

_sc_call = functools.partial(
    pl.kernel,
    out_type=jax.ShapeDtypeStruct((B, T, D), jnp.float32),
    mesh=_mesh,
    scratch_types=(
        [pltpu.VMEM((NP, BTR, D), jnp.float32),
         pltpu.VMEM((NX, B, BTR, D), jnp.float32)]
        + [pltpu.SemaphoreType.DMA] * (NP + 2 * NX)
    ),
)(_sc_add)


def kernel(x, pe_table):
    return _sc_call(x, pe_table[:T])
